# baseline (speedup 1.0000x reference)
import jax
import jax.numpy as jnp
from jax.experimental import pallas as pl

N = 10000
G = 64


def _gat(x, src, dst, W, a_src, a_dst, b):
    h = x @ W
    al_s = h @ a_src
    al_d = h @ a_dst
    e = jax.nn.leaky_relu(al_s[src] + al_d[dst], negative_slope=0.2)
    e_max = jax.ops.segment_max(e, dst, num_segments=N)
    e_max = jnp.where(jnp.isfinite(e_max), e_max, 0.0)
    ex = jnp.exp(e - e_max[dst])
    denom = jax.ops.segment_sum(ex, dst, num_segments=N)
    alpha = ex / (denom[dst] + 1e-16)
    msg = alpha[:, None] * h[src]
    return jax.ops.segment_sum(msg, dst, num_segments=N) + b


def kernel(x, edge_index, batch, W1, a_src1, a_dst1, b1, W2, a_src2, a_dst2, b2,
           W3, a_src3, a_dst3, b3, W4, a_src4, a_dst4, b4, Wfc, bfc):
    src = edge_index[0]
    dst = edge_index[1]
    h = jax.nn.relu(_gat(x, src, dst, W1, a_src1, a_dst1, b1))
    h = jax.nn.relu(_gat(h, src, dst, W2, a_src2, a_dst2, b2))
    h = jax.nn.relu(_gat(h, src, dst, W3, a_src3, a_dst3, b3))
    h = jax.nn.relu(_gat(h, src, dst, W4, a_src4, a_dst4, b4))
    sums = jax.ops.segment_sum(h, batch, num_segments=G)
    counts = jax.ops.segment_sum(jnp.ones((N,), jnp.float32), batch, num_segments=G)
    pooled = sums / jnp.maximum(counts, 1.0)[:, None]
    return pooled @ Wfc + bfc


# speedup vs baseline: 1.4373x; 1.4373x over previous
import functools

import jax
import jax.numpy as jnp
from jax.experimental import pallas as pl
from jax.experimental.pallas import tpu as pltpu

N = 10000
E = 320000
H = 128
G = 64
RB = 1000
NBLK = N // RB


def _head_first_body(x_ref, w_ref, asrc_ref, adst_ref,
                     h_ref, als_ref, ald_ref, maxs_ref):
    i = pl.program_id(0)
    xb = x_ref[...]
    h = jnp.dot(xb, w_ref[...], preferred_element_type=jnp.float32)
    h_ref[...] = h
    als = jnp.dot(h, asrc_ref[...], preferred_element_type=jnp.float32)
    ald = jnp.dot(h, adst_ref[...], preferred_element_type=jnp.float32)
    als_ref[...] = als
    ald_ref[...] = ald

    @pl.when(i == 0)
    def _():
        maxs_ref[...] = jnp.full((1, 1), -jnp.inf, jnp.float32)
    maxs_ref[...] = jnp.maximum(maxs_ref[...], jnp.max(als).reshape(1, 1))


def _head_rest_body(p0_ref, p1_ref, b_ref, w_ref, asrc_ref, adst_ref,
                    h_ref, als_ref, ald_ref, maxs_ref):
    i = pl.program_id(0)
    xb = jax.nn.relu(p0_ref[...] + p1_ref[...] + b_ref[...])
    h = jnp.dot(xb, w_ref[...], preferred_element_type=jnp.float32)
    h_ref[...] = h
    als = jnp.dot(h, asrc_ref[...], preferred_element_type=jnp.float32)
    ald = jnp.dot(h, adst_ref[...], preferred_element_type=jnp.float32)
    als_ref[...] = als
    ald_ref[...] = ald

    @pl.when(i == 0)
    def _():
        maxs_ref[...] = jnp.full((1, 1), -jnp.inf, jnp.float32)
    maxs_ref[...] = jnp.maximum(maxs_ref[...], jnp.max(als).reshape(1, 1))


_HEAD_OUT = (
    jax.ShapeDtypeStruct((N, H), jnp.float32),
    jax.ShapeDtypeStruct((N, 1), jnp.float32),
    jax.ShapeDtypeStruct((N, 1), jnp.float32),
    jax.ShapeDtypeStruct((1, 1), jnp.float32),
)
_HEAD_OUT_SPECS = (
    pl.BlockSpec((RB, H), lambda i: (i, 0)),
    pl.BlockSpec((RB, 1), lambda i: (i, 0)),
    pl.BlockSpec((RB, 1), lambda i: (i, 0)),
    pl.BlockSpec((1, 1), lambda i: (0, 0)),
)
_W_SPEC = pl.BlockSpec((H, H), lambda i: (0, 0))
_A_SPEC = pl.BlockSpec((H, 1), lambda i: (0, 0))
_X_SPEC = pl.BlockSpec((RB, H), lambda i: (i, 0))
_B_SPEC = pl.BlockSpec((1, H), lambda i: (0, 0))


_head_first = pl.pallas_call(
    _head_first_body,
    grid=(NBLK,),
    in_specs=[_X_SPEC, _W_SPEC, _A_SPEC, _A_SPEC],
    out_specs=_HEAD_OUT_SPECS,
    out_shape=_HEAD_OUT,
)

_head_rest = pl.pallas_call(
    _head_rest_body,
    grid=(NBLK,),
    in_specs=[_X_SPEC, _X_SPEC, _B_SPEC, _W_SPEC, _A_SPEC, _A_SPEC],
    out_specs=_HEAD_OUT_SPECS,
    out_shape=_HEAD_OUT,
)


def _pool_body(p0_ref, p1_ref, b_ref, batch_ref, wfc_ref, bfc_ref,
               out_ref, sums_ref, counts_ref):
    i = pl.program_id(0)

    @pl.when(i == 0)
    def _():
        sums_ref[...] = jnp.zeros_like(sums_ref)
        counts_ref[...] = jnp.zeros_like(counts_ref)

    xb = jax.nn.relu(p0_ref[...] + p1_ref[...] + b_ref[...])
    onehot = (batch_ref[...] == jax.lax.broadcasted_iota(jnp.int32, (1, G), 1)
              ).astype(jnp.float32)
    sums_ref[...] += jax.lax.dot_general(
        onehot, xb, (((0,), (0,)), ((), ())), preferred_element_type=jnp.float32)
    counts_ref[...] += jax.lax.dot_general(
        onehot, jnp.ones((RB, 1), jnp.float32), (((0,), (0,)), ((), ())),
        preferred_element_type=jnp.float32)

    @pl.when(i == NBLK - 1)
    def _():
        num = jnp.dot(sums_ref[...], wfc_ref[...],
                      preferred_element_type=jnp.float32)
        out_ref[...] = num / jnp.maximum(counts_ref[...], 1.0) + bfc_ref[...]


_pool = pl.pallas_call(
    _pool_body,
    grid=(NBLK,),
    in_specs=[_X_SPEC, _X_SPEC, _B_SPEC,
              pl.BlockSpec((RB, 1), lambda i: (i, 0)),
              pl.BlockSpec((H, 1), lambda i: (0, 0)),
              pl.BlockSpec((1, 1), lambda i: (0, 0))],
    out_specs=pl.BlockSpec((G, 1), lambda i: (0, 0)),
    out_shape=jax.ShapeDtypeStruct((G, 1), jnp.float32),
    scratch_shapes=[pltpu.VMEM((G, H), jnp.float32),
                    pltpu.VMEM((G, 1), jnp.float32)],
)


def _edge_agg_xla(h, als, ald, maxs, src, dst):
    t = als[src] + ald[dst]
    e = jnp.where(t > 0, t, 0.2 * t)
    bnd = maxs + ald[dst]
    mb = jnp.where(bnd > 0, bnd, 0.2 * bnd)
    ex = jnp.exp(e - mb)
    denom = jax.ops.segment_sum(ex, dst, num_segments=N)
    alpha = ex / (denom[dst] + 1e-16)
    msg = alpha[:, None] * h[src]
    part = jax.ops.segment_sum(msg, dst, num_segments=N)
    return part, jnp.zeros_like(part)


def kernel(x, edge_index, batch, W1, a_src1, a_dst1, b1, W2, a_src2, a_dst2, b2,
           W3, a_src3, a_dst3, b3, W4, a_src4, a_dst4, b4, Wfc, bfc):
    src = edge_index[0]
    dst = edge_index[1]
    layers = [(W1, a_src1, a_dst1, b1), (W2, a_src2, a_dst2, b2),
              (W3, a_src3, a_dst3, b3), (W4, a_src4, a_dst4, b4)]

    p0 = p1 = None
    bprev = None
    for li, (W, asrc, adst, b) in enumerate(layers):
        if li == 0:
            h, als, ald, maxs = _head_first(x, W, asrc.reshape(H, 1),
                                            adst.reshape(H, 1))
        else:
            h, als, ald, maxs = _head_rest(p0, p1, bprev.reshape(1, H), W,
                                           asrc.reshape(H, 1),
                                           adst.reshape(H, 1))
        p0, p1 = _edge_agg_xla(h, als.reshape(N), ald.reshape(N),
                               maxs.reshape(()), src, dst)
        bprev = b

    return _pool(p0, p1, bprev.reshape(1, H), batch.reshape(N, 1),
                 Wfc, bfc.reshape(1, 1))


# speedup vs baseline: 12.9314x; 8.9969x over previous
import functools

import jax
import jax.numpy as jnp
from jax import lax
from jax.experimental import pallas as pl
from jax.experimental.pallas import tpu as pltpu
from jax.experimental.pallas import tpu_sc as plsc

N = 10000
E = 320000
H = 128
G = 64
RB = 1000
NBLK = N // RB


def _head_first_body(x_ref, w_ref, asrc_ref, adst_ref,
                     h_ref, als_ref, ald_ref, maxs_ref):
    i = pl.program_id(0)
    xb = x_ref[...]
    h = jnp.dot(xb, w_ref[...], preferred_element_type=jnp.float32)
    h_ref[...] = h
    als = jnp.dot(h, asrc_ref[...], preferred_element_type=jnp.float32)
    ald = jnp.dot(h, adst_ref[...], preferred_element_type=jnp.float32)
    als_ref[...] = als
    ald_ref[...] = ald

    @pl.when(i == 0)
    def _():
        maxs_ref[...] = jnp.full((1, 1), -jnp.inf, jnp.float32)
    maxs_ref[...] = jnp.maximum(maxs_ref[...], jnp.max(als).reshape(1, 1))


def _head_rest_body(p0_ref, p1_ref, d_ref, b_ref, w_ref, asrc_ref,
                    adst_ref, h_ref, als_ref, ald_ref, maxs_ref):
    i = pl.program_id(0)
    den = d_ref[...] + 1e-16
    b = b_ref[...]
    x0 = jax.nn.relu(p0_ref[...] / den + b[:, :64])
    x1 = jax.nn.relu(p1_ref[...] / den + b[:, 64:])
    h = (jnp.dot(x0, w_ref[0:64, :], preferred_element_type=jnp.float32)
         + jnp.dot(x1, w_ref[64:128, :], preferred_element_type=jnp.float32))
    h_ref[...] = h
    als = jnp.dot(h, asrc_ref[...], preferred_element_type=jnp.float32)
    ald = jnp.dot(h, adst_ref[...], preferred_element_type=jnp.float32)
    als_ref[...] = als
    ald_ref[...] = ald

    @pl.when(i == 0)
    def _():
        maxs_ref[...] = jnp.full((1, 1), -jnp.inf, jnp.float32)
    maxs_ref[...] = jnp.maximum(maxs_ref[...], jnp.max(als).reshape(1, 1))


_HEAD_OUT = (
    jax.ShapeDtypeStruct((N, H), jnp.float32),
    jax.ShapeDtypeStruct((N, 1), jnp.float32),
    jax.ShapeDtypeStruct((N, 1), jnp.float32),
    jax.ShapeDtypeStruct((1, 1), jnp.float32),
)
_HEAD_OUT_SPECS = (
    pl.BlockSpec((RB, H), lambda i: (i, 0)),
    pl.BlockSpec((RB, 1), lambda i: (i, 0)),
    pl.BlockSpec((RB, 1), lambda i: (i, 0)),
    pl.BlockSpec((1, 1), lambda i: (0, 0)),
)
_W_SPEC = pl.BlockSpec((H, H), lambda i: (0, 0))
_A_SPEC = pl.BlockSpec((H, 1), lambda i: (0, 0))
_X_SPEC = pl.BlockSpec((RB, H), lambda i: (i, 0))
_P_SPEC = pl.BlockSpec((RB, 64), lambda i: (i, 0))
_B_SPEC = pl.BlockSpec((1, H), lambda i: (0, 0))
_D_SPEC = pl.BlockSpec((RB, 1), lambda i: (i, 0))


_head_first = pl.pallas_call(
    _head_first_body,
    grid=(NBLK,),
    in_specs=[_X_SPEC, _W_SPEC, _A_SPEC, _A_SPEC],
    out_specs=_HEAD_OUT_SPECS,
    out_shape=_HEAD_OUT,
)

_head_rest = pl.pallas_call(
    _head_rest_body,
    grid=(NBLK,),
    in_specs=[_P_SPEC, _P_SPEC, _D_SPEC, _B_SPEC,
              _W_SPEC, _A_SPEC, _A_SPEC],
    out_specs=_HEAD_OUT_SPECS,
    out_shape=_HEAD_OUT,
)


def _pool_body(p0_ref, p1_ref, d_ref, b_ref, batch_ref, wfc_ref,
               bfc_ref, out_ref, sums_ref, counts_ref):
    i = pl.program_id(0)

    @pl.when(i == 0)
    def _():
        sums_ref[...] = jnp.zeros_like(sums_ref)
        counts_ref[...] = jnp.zeros_like(counts_ref)

    den = d_ref[...] + 1e-16
    b = b_ref[...]
    x0 = jax.nn.relu(p0_ref[...] / den + b[:, :64])
    x1 = jax.nn.relu(p1_ref[...] / den + b[:, 64:])
    onehot = (batch_ref[...] == jax.lax.broadcasted_iota(jnp.int32, (1, G), 1)
              ).astype(jnp.float32)
    sums_ref[:, 0:64] += jax.lax.dot_general(
        onehot, x0, (((0,), (0,)), ((), ())), preferred_element_type=jnp.float32)
    sums_ref[:, 64:128] += jax.lax.dot_general(
        onehot, x1, (((0,), (0,)), ((), ())), preferred_element_type=jnp.float32)
    counts_ref[...] += jax.lax.dot_general(
        onehot, jnp.ones((RB, 1), jnp.float32), (((0,), (0,)), ((), ())),
        preferred_element_type=jnp.float32)

    @pl.when(i == NBLK - 1)
    def _():
        num = jnp.dot(sums_ref[...], wfc_ref[...],
                      preferred_element_type=jnp.float32)
        out_ref[...] = num / jnp.maximum(counts_ref[...], 1.0) + bfc_ref[...]


_pool = pl.pallas_call(
    _pool_body,
    grid=(NBLK,),
    in_specs=[_P_SPEC, _P_SPEC, _D_SPEC, _B_SPEC,
              pl.BlockSpec((RB, 1), lambda i: (i, 0)),
              pl.BlockSpec((H, 1), lambda i: (0, 0)),
              pl.BlockSpec((1, 1), lambda i: (0, 0))],
    out_specs=pl.BlockSpec((G, 1), lambda i: (0, 0)),
    out_shape=jax.ShapeDtypeStruct((G, 1), jnp.float32),
    scratch_shapes=[pltpu.VMEM((G, H), jnp.float32),
                    pltpu.VMEM((G, 1), jnp.float32)],
)


NT = 16
EPT = E // NT
NCHUNK = EPT // 16
DEPTH = 4
RPT = N // NT
WC = 80


def _sc_agg_body(h2_hbm, als_hbm, ald_hbm, maxs_hbm, src_hbm, dst_hbm,
                 outr_hbm,
                 src_v, dst_v, als_v, ald_v, maxs_v, scale_v,
                 dvtmp, rowtmp, zbuf,
                 rows0, rows1, rows2, rows3,
                 acc0, acc1, acc2, acc3,
                 gsem0, gsem1, gsem2, gsem3,
                 out_sh):
    c = lax.axis_index("c")
    s = lax.axis_index("s")
    ebase = s * EPT
    coff = c * N

    rows = (rows0, rows1, rows2, rows3)
    acc = (acc0, acc1, acc2, acc3)
    gsem = (gsem0, gsem1, gsem2, gsem3)

    pltpu.sync_copy(als_hbm, als_v)
    pltpu.sync_copy(ald_hbm, ald_v)
    pltpu.sync_copy(maxs_hbm, maxs_v)
    pltpu.sync_copy(src_hbm.at[pl.ds(ebase, EPT)], src_v)
    pltpu.sync_copy(dst_hbm.at[pl.ds(ebase, EPT)], dst_v)

    zero16 = jnp.zeros((16,), jnp.float32)

    def _zzb(i, _):
        for j in range(WC // 16):
            zbuf[i, pl.ds(j * 16, 16)] = zero16
        return 0
    lax.fori_loop(0, 125, _zzb, 0)

    for j in range(5):
        pltpu.sync_copy(zbuf, out_sh.at[pl.ds(s * RPT + j * 125, 125)])
    plsc.subcore_barrier()

    maxs = maxs_v[...]
    iota16 = lax.iota(jnp.int32, 16)
    unit = jnp.where(iota16 == 0, 1.0, 0.0)

    def _issue(chunk, b):
        sv = src_v[pl.ds(chunk * 16, 16)] + coff
        pltpu.async_copy(h2_hbm.at[sv], rows[b], gsem[b])

    def _process(chunk, b):
        sv = src_v[pl.ds(chunk * 16, 16)]
        dv = dst_v[pl.ds(chunk * 16, 16)]
        sval = plsc.load_gather(als_v, [sv])
        dval = plsc.load_gather(ald_v, [dv])
        t = sval + dval
        e = jnp.where(t > 0, t, 0.2 * t)
        bnd = maxs + dval
        mb = jnp.where(bnd > 0, bnd, 0.2 * bnd)
        ex = jnp.exp(e - mb)
        scale_v[...] = ex
        pltpu.make_async_copy(h2_hbm.at[sv + coff], rows[b], gsem[b]).wait()
        for ee in range(16):
            se = plsc.load_gather(scale_v, [jnp.full((16,), ee, jnp.int32)])
            for j in range(4):
                acc[b][ee, pl.ds(j * 16, 16)] = (
                    rows[b][ee, pl.ds(j * 16, 16)] * se)
            acc[b][ee, pl.ds(64, 16)] = se * unit

        kd, _ = plsc.sort_key_val(dv, dv)
        dvtmp[pl.ds(0, 16)] = kd
        dvtmp[pl.ds(16, 16)] = kd
        eqd = jnp.where(kd == dvtmp[pl.ds(1, 16)], 1, 0)
        ks, _ = plsc.sort_key_val(sv, sv)
        dvtmp[pl.ds(0, 16)] = ks
        dvtmp[pl.ds(16, 16)] = ks
        eqs = jnp.where(ks == dvtmp[pl.ds(1, 16)], 1, 0)
        anydup = jnp.max(eqd | eqs)

        @pl.when(anydup == 0)
        def _():
            pltpu.sync_copy(acc[b], out_sh.at[dv], add=True)

        @pl.when(anydup != 0)
        def _():
            dvtmp[pl.ds(0, 16)] = sv
            dvtmp[pl.ds(16, 16)] = sv
            dups = jnp.zeros((16,), jnp.int32)
            for m in range(1, 9):
                dups = dups | jnp.where(sv == dvtmp[pl.ds(m, 16)], 1, 0)
            dvtmp[pl.ds(0, 16)] = dv
            dvtmp[pl.ds(16, 16)] = dv
            dupd = jnp.zeros((16,), jnp.int32)
            for m in range(1, 9):
                dupd = dupd | jnp.where(dv == dvtmp[pl.ds(m, 16)], 1, 0)
            dupany = dups | dupd

            idx0 = jnp.where(dupany != 0, N + iota16, dv)
            pltpu.sync_copy(acc[b], out_sh.at[idx0], add=True)

            for ee in range(16):
                flag_s = jnp.max(jnp.where(iota16 == ee, dups, 0))
                flag_a = jnp.max(jnp.where(iota16 == ee, dupany, 0))

                @pl.when(flag_s != 0)
                def _():
                    srow = jnp.max(jnp.where(iota16 == ee, sv, 0)) + coff
                    pltpu.sync_copy(h2_hbm.at[pl.ds(srow, 1)], rowtmp)
                    se = plsc.load_gather(
                        scale_v, [jnp.full((16,), ee, jnp.int32)])
                    for j in range(4):
                        acc[b][ee, pl.ds(j * 16, 16)] = (
                            rowtmp[0, pl.ds(j * 16, 16)] * se)

                @pl.when(flag_a != 0)
                def _():
                    si = jnp.where(iota16 == ee, dv, N + iota16)
                    pltpu.sync_copy(acc[b], out_sh.at[si], add=True)

        nxt = chunk + DEPTH

        @pl.when(nxt < NCHUNK)
        def _():
            _issue(nxt, b)

    for b in range(DEPTH):
        _issue(b, b)

    def _loop(k, _):
        for b in range(DEPTH):
            _process(k * DEPTH + b, b)
        return 0
    lax.fori_loop(0, NCHUNK // DEPTH, _loop, 0)
    for b in range(NCHUNK % DEPTH):
        _process((NCHUNK // DEPTH) * DEPTH + b, b)

    plsc.subcore_barrier()

    @pl.when(s < 10)
    def _():
        pltpu.sync_copy(out_sh.at[pl.ds(s * 1000, 1000)],
                        outr_hbm.at[pl.ds(c * N + s * 1000, 1000)])


_sc_agg = functools.partial(
    pl.kernel,
    out_type=jax.ShapeDtypeStruct((2 * N, WC), jnp.float32),
    mesh=plsc.VectorSubcoreMesh(core_axis_name="c", subcore_axis_name="s"),
    compiler_params=pltpu.CompilerParams(
        needs_layout_passes=False, use_tc_tiling_on_sc=False),
    scratch_types=[
        pltpu.VMEM((EPT,), jnp.int32),
        pltpu.VMEM((EPT,), jnp.int32),
        pltpu.VMEM((N,), jnp.float32),
        pltpu.VMEM((N,), jnp.float32),
        pltpu.VMEM((16,), jnp.float32),
        pltpu.VMEM((16,), jnp.float32),
        pltpu.VMEM((32,), jnp.int32),
        pltpu.VMEM((1, 64), jnp.float32),
        pltpu.VMEM((125, WC), jnp.float32),
        pltpu.VMEM((16, 64), jnp.float32),
        pltpu.VMEM((16, 64), jnp.float32),
        pltpu.VMEM((16, 64), jnp.float32),
        pltpu.VMEM((16, 64), jnp.float32),
        pltpu.VMEM((16, WC), jnp.float32),
        pltpu.VMEM((16, WC), jnp.float32),
        pltpu.VMEM((16, WC), jnp.float32),
        pltpu.VMEM((16, WC), jnp.float32),
        pltpu.SemaphoreType.DMA,
        pltpu.SemaphoreType.DMA,
        pltpu.SemaphoreType.DMA,
        pltpu.SemaphoreType.DMA,
        pltpu.VMEM_SHARED((N + 16, WC), jnp.float32),
    ],
)(_sc_agg_body)


def _edge_agg(h, als, ald, maxs, src, dst):
    h2 = jnp.concatenate([h[:, :64], h[:, 64:],
                          jnp.zeros((16, 64), jnp.float32)], axis=0)
    maxs16 = jnp.broadcast_to(maxs.reshape(1), (16,))
    outr = _sc_agg(h2, als.reshape(N), ald.reshape(N), maxs16, src, dst)
    p0 = outr[:N, 0:64]
    p1 = outr[N:, 0:64]
    den = outr[:N, 64:65]
    return p0, p1, den


def kernel(x, edge_index, batch, W1, a_src1, a_dst1, b1, W2, a_src2, a_dst2, b2,
           W3, a_src3, a_dst3, b3, W4, a_src4, a_dst4, b4, Wfc, bfc):
    layers = [(W1, a_src1, a_dst1, b1), (W2, a_src2, a_dst2, b2),
              (W3, a_src3, a_dst3, b3), (W4, a_src4, a_dst4, b4)]
    src = edge_index[0]
    dst = edge_index[1]

    p0 = p1 = den = None
    bprev = None
    for li, (W, asrc, adst, b) in enumerate(layers):
        if li == 0:
            h, als, ald, maxs = _head_first(x, W, asrc.reshape(H, 1),
                                            adst.reshape(H, 1))
        else:
            h, als, ald, maxs = _head_rest(p0, p1, den, bprev.reshape(1, H),
                                           W, asrc.reshape(H, 1),
                                           adst.reshape(H, 1))
        p0, p1, den = _edge_agg(h, als, ald, maxs, src, dst)
        bprev = b

    return _pool(p0, p1, den, bprev.reshape(1, H), batch.reshape(N, 1),
                 Wfc, bfc.reshape(1, 1))
